# baseline (speedup 1.0000x reference)
import functools

import jax
import jax.numpy as jnp
from jax import lax
from jax.experimental import pallas as pl
from jax.experimental.pallas import tpu as pltpu
from jax.experimental.pallas import tpu_sc as plsc


def _build_lookup(N, V, D, L):
    info = plsc.get_sparse_core_info()
    nc, ns = info.num_cores, info.num_subcores
    nw = nc * ns
    per_w = N // nw
    assert per_w * nw == N
    CHUNK = L
    n_chunks = per_w // CHUNK
    assert n_chunks * CHUNK == per_w
    G0, G1 = 128, CHUNK - 128

    mesh = plsc.VectorSubcoreMesh(core_axis_name="c", subcore_axis_name="s")

    @functools.partial(
        pl.kernel,
        out_type=jax.ShapeDtypeStruct((N, D), jnp.float32),
        mesh=mesh,
        compiler_params=pltpu.CompilerParams(use_tc_tiling_on_sc=False),
        scratch_types=[
            pltpu.VMEM((CHUNK,), jnp.int32),
            pltpu.VMEM((CHUNK, D), jnp.float32),
            pltpu.VMEM((L, D), jnp.float32),
            pltpu.SemaphoreType.DMA,
        ],
    )
    def emb(x_hbm, tok_hbm, pos_hbm, out_hbm, idx_v, rows_v, pos_v, sem):
        wid = lax.axis_index("s") * nc + lax.axis_index("c")
        pltpu.sync_copy(pos_hbm, pos_v)
        base0 = wid * per_w

        def chunk_body(c, carry):
            base = base0 + c * CHUNK
            pltpu.sync_copy(x_hbm.at[pl.ds(base, CHUNK)], idx_v)
            pltpu.async_copy(
                tok_hbm.at[idx_v.at[pl.ds(0, G0)]],
                rows_v.at[pl.ds(0, G0)], sem).wait()
            pltpu.async_copy(
                tok_hbm.at[idx_v.at[pl.ds(G0, G1)]],
                rows_v.at[pl.ds(G0, G1)], sem).wait()

            def add_body(r, carry2):
                for d in range(D // 16):
                    sl = pl.ds(d * 16, 16)
                    rows_v[r, sl] = rows_v[r, sl] + pos_v[r, sl]
                return carry2

            lax.fori_loop(0, CHUNK, add_body, 0)
            pltpu.sync_copy(rows_v, out_hbm.at[pl.ds(base, CHUNK)])
            return carry

        lax.fori_loop(0, n_chunks, chunk_body, 0)

    return emb


def kernel(x, token_table, pos_table):
    B, L = x.shape
    V, D = token_table.shape
    N = B * L
    x_flat = x.reshape(N).astype(jnp.int32)
    emb = _build_lookup(N, V, D, L)
    out = emb(x_flat, token_table, pos_table)
    return out.reshape(B, L, D)


# speedup vs baseline: 1.2246x; 1.2246x over previous
import functools

import jax
import jax.numpy as jnp
from jax import lax
from jax.experimental import pallas as pl
from jax.experimental.pallas import tpu as pltpu
from jax.experimental.pallas import tpu_sc as plsc


def _build_lookup(N, V, D, L, CHUNK):
    info = plsc.get_sparse_core_info()
    nc, ns = info.num_cores, info.num_subcores
    nw = nc * ns
    per_w = N // nw
    assert per_w * nw == N
    assert CHUNK % L == 0
    n_chunks = per_w // CHUNK
    assert n_chunks * CHUNK == per_w and n_chunks % 2 == 0
    splits = []
    off = 0
    while off < CHUNK:
        g = min(128, CHUNK - off)
        splits.append((off, g))
        off += g
    LANES = D // 16

    mesh = plsc.VectorSubcoreMesh(core_axis_name="c", subcore_axis_name="s")

    @functools.partial(
        pl.kernel,
        out_type=jax.ShapeDtypeStruct((N, D), jnp.float32),
        mesh=mesh,
        compiler_params=pltpu.CompilerParams(use_tc_tiling_on_sc=False),
        scratch_types=[
            pltpu.VMEM((per_w,), jnp.int32),
            pltpu.VMEM((CHUNK, D), jnp.float32),
            pltpu.VMEM((CHUNK, D), jnp.float32),
            pltpu.VMEM((L, D), jnp.float32),
            pltpu.SemaphoreType.DMA,
            pltpu.SemaphoreType.DMA,
            pltpu.SemaphoreType.DMA,
            pltpu.SemaphoreType.DMA,
        ],
    )
    def emb(x_hbm, tok_hbm, pos_hbm, out_hbm,
            idx_all, rows0, rows1, pos_v, g0, g1, s0, s1):
        wid = lax.axis_index("s") * nc + lax.axis_index("c")
        base0 = wid * per_w
        pltpu.sync_copy(x_hbm.at[pl.ds(base0, per_w)], idx_all)
        pltpu.sync_copy(pos_hbm, pos_v)
        bufs = ((rows0, g0, s0), (rows1, g1, s1))

        def issue(g, rowsb, gsem):
            for off, gl in splits:
                pltpu.async_copy(
                    tok_hbm.at[idx_all.at[pl.ds(g * CHUNK + off, gl)]],
                    rowsb.at[pl.ds(off, gl)], gsem)

        def wait_gather(g, rowsb, gsem):
            for off, gl in splits:
                pltpu.make_async_copy(
                    tok_hbm.at[idx_all.at[pl.ds(g * CHUNK + off, gl)]],
                    rowsb.at[pl.ds(off, gl)], gsem).wait()

        def add_pos(rowsb):
            def add_body(r, carry):
                p = lax.rem(r, L)
                for d in range(LANES):
                    sl = pl.ds(d * 16, 16)
                    rowsb[r, sl] = rowsb[r, sl] + pos_v[p, sl]
                return carry
            lax.fori_loop(0, CHUNK, add_body, 0)

        issue(0, rows0, g0)

        def pair_body(t, carry):
            s = t * 2
            for j in range(2):
                g = s + j
                rowsb, gsem, ssem = bufs[j]
                orows, ogsem, ossem = bufs[1 - j]

                @pl.when(g + 1 < n_chunks)
                def _issue_next():
                    @pl.when(g >= 1)
                    def _drain_store():
                        pltpu.make_async_copy(
                            orows, out_hbm.at[pl.ds(0, CHUNK)], ossem).wait()
                    issue(g + 1, orows, ogsem)

                wait_gather(g, rowsb, gsem)
                add_pos(rowsb)
                pltpu.async_copy(
                    rowsb, out_hbm.at[pl.ds(base0 + g * CHUNK, CHUNK)], ssem)
            return carry

        lax.fori_loop(0, n_chunks // 2, pair_body, 0)
        pltpu.make_async_copy(rows0, out_hbm.at[pl.ds(0, CHUNK)], s0).wait()
        pltpu.make_async_copy(rows1, out_hbm.at[pl.ds(0, CHUNK)], s1).wait()

    return emb


def kernel(x, token_table, pos_table):
    B, L = x.shape
    V, D = token_table.shape
    N = B * L
    x_flat = x.reshape(N).astype(jnp.int32)
    emb = _build_lookup(N, V, D, L, CHUNK=200)
    out = emb(x_flat, token_table, pos_table)
    return out.reshape(B, L, D)
